# baseline (speedup 1.0000x reference)
import functools

import jax
import jax.numpy as jnp
from jax import lax
from jax.experimental import pallas as pl
from jax.experimental.pallas import tpu as pltpu
from jax.experimental.pallas import tpu_sc as plsc

_NC = 2
_NS = 16
_NW = _NC * _NS
_LANES = 16
_TROW = 128
_NG = 2
_NT = 4


def _pos_embedding(emb_dim, seq_len):
    pos = jnp.arange(seq_len, dtype=jnp.float32)[:, None]
    i = jnp.arange(emb_dim, dtype=jnp.float32)[None, :]
    angle_rates = 1.0 / jnp.power(10000.0, (2.0 * jnp.floor(i / 2.0)) / float(emb_dim))
    angles = pos * angle_rates
    even = (jnp.arange(emb_dim)[None, :] % 2) == 0
    return jnp.where(even, jnp.sin(angles), jnp.cos(angles)).astype(jnp.float32)


def kernel(inputs, token_embeddings):
    batch, seq_len = inputs.shape
    vocab, emb_dim = token_embeddings.shape
    bpw = batch // _NW
    nblk = bpw // _LANES

    pe = _pos_embedding(emb_dim, seq_len)
    idx_t = inputs.T

    mesh = plsc.VectorSubcoreMesh(core_axis_name="c", subcore_axis_name="s")

    @functools.partial(
        pl.kernel,
        out_type=jax.ShapeDtypeStruct(
            (seq_len, batch // _TROW, emb_dim // 8, 8, _TROW), jnp.float32),
        mesh=mesh,
        compiler_params=pltpu.CompilerParams(
            use_tc_tiling_on_sc=False, needs_layout_passes=False),
        scratch_types=[
            pltpu.VMEM((seq_len, bpw), jnp.int32),
            pltpu.VMEM((seq_len, emb_dim), jnp.float32),
            [pltpu.VMEM((bpw, emb_dim), jnp.float32) for _ in range(_NG)],
            [pltpu.VMEM((emb_dim // 8, 8, _TROW), jnp.float32)
             for _ in range(_NT)],
            [pltpu.SemaphoreType.DMA for _ in range(_NG)],
            [pltpu.SemaphoreType.DMA for _ in range(_NT)],
        ],
    )
    def emb(idx_hbm, table_hbm, pe_hbm, out_hbm,
            idx_v, pe_v, gbufs, tbufs, gsems, ssems):
        wid = lax.axis_index("s") * _NC + lax.axis_index("c")
        b0 = wid * bpw
        pltpu.sync_copy(idx_hbm.at[:, pl.ds(b0, bpw)], idx_v)
        pltpu.sync_copy(pe_hbm, pe_v)

        rvecs = [lax.iota(jnp.int32, _LANES) + (l * _LANES)
                 for l in range(nblk)]

        def issue_gather(s, g):
            pltpu.async_copy(table_hbm.at[idx_v.at[s]], gbufs[g], gsems[g])

        def wait_gather(s, g):
            pltpu.make_async_copy(
                table_hbm.at[idx_v.at[s]], gbufs[g], gsems[g]).wait()

        def out_slice(s):
            return out_hbm.at[s, wid]

        def issue_store(s, t):
            pltpu.async_copy(tbufs[t], out_slice(s), ssems[t])

        def wait_store(s, t):
            pltpu.make_async_copy(tbufs[t], out_slice(s), ssems[t]).wait()

        def transpose_add(s, g, t):
            gbuf = gbufs[g]
            tbuf = tbufs[t]
            svec = jnp.full((_LANES,), s, jnp.int32)

            def feat_body(d, carry):
                dvec = jnp.full((_LANES,), d, jnp.int32)
                pe_sd = plsc.load_gather(pe_v, [svec, dvec])
                dhi = jnp.right_shift(d, 3)
                dlo = jnp.bitwise_and(d, 7)
                vs = [plsc.load_gather(gbuf, [rvecs[l], dvec])
                      for l in range(nblk)]
                for l in range(nblk):
                    tbuf[dhi, dlo, pl.ds(l * _LANES, _LANES)] = vs[l] + pe_sd
                return carry

            lax.fori_loop(0, emb_dim, feat_body, 0, unroll=4)

        for g in range(_NG):
            issue_gather(g, g)

        def body(jj, carry):
            for k in range(_NT):
                s = _NT * jj + k
                g = k % _NG
                t = k
                wait_gather(s, g)

                @pl.when(s >= _NT)
                def _():
                    wait_store(s - _NT, t)

                transpose_add(s, g, t)
                issue_store(s, t)

                @pl.when(s + _NG < seq_len)
                def _():
                    issue_gather(s + _NG, g)
            return carry

        lax.fori_loop(0, seq_len // _NT, body, 0)

        for t in range(_NT):
            wait_store(seq_len - _NT + t, t)

    out5 = emb(idx_t, token_embeddings, pe)
    out = jnp.transpose(out5, (1, 4, 0, 2, 3))
    return out.reshape(batch, seq_len, emb_dim)


# speedup vs baseline: 1.3137x; 1.3137x over previous
import functools

import jax
import jax.numpy as jnp
from jax import lax
from jax.experimental import pallas as pl
from jax.experimental.pallas import tpu as pltpu
from jax.experimental.pallas import tpu_sc as plsc

_EMB_DIM = 64
_SEQ_LEN = 200
_NC = 2
_NS = 16
_NW = _NC * _NS
_CHUNK = 128
_LANES = 16
_NB = 4
_RUNROLL = 16


def _pos_embedding(emb_dim, seq_len):
    pos = jnp.arange(seq_len, dtype=jnp.float32)[:, None]
    i = jnp.arange(emb_dim, dtype=jnp.float32)[None, :]
    angle_rates = 1.0 / jnp.power(10000.0, (2.0 * jnp.floor(i / 2.0)) / float(emb_dim))
    angles = pos * angle_rates
    even = (jnp.arange(emb_dim)[None, :] % 2) == 0
    return jnp.where(even, jnp.sin(angles), jnp.cos(angles)).astype(jnp.float32)


def kernel(inputs, token_embeddings):
    batch, seq_len = inputs.shape
    _, emb_dim = token_embeddings.shape
    rows = batch * seq_len
    rpw = rows // _NW
    nch = rpw // _CHUNK
    nvec = emb_dim // _LANES

    pe = _pos_embedding(emb_dim, seq_len)
    pe2 = jnp.concatenate([pe, pe], axis=0)
    idx = inputs.reshape(_NW, nch, _CHUNK)

    mesh = plsc.VectorSubcoreMesh(core_axis_name="c", subcore_axis_name="s")

    @functools.partial(
        pl.kernel,
        out_type=jax.ShapeDtypeStruct((rows, emb_dim), jnp.float32),
        mesh=mesh,
        compiler_params=pltpu.CompilerParams(use_tc_tiling_on_sc=False),
        scratch_types=[
            pltpu.VMEM((nch, _CHUNK), jnp.int32),
            pltpu.VMEM((2 * seq_len, emb_dim), jnp.float32),
            [pltpu.VMEM((_CHUNK, emb_dim), jnp.float32) for _ in range(_NB)],
            [pltpu.SemaphoreType.DMA for _ in range(_NB)],
            [pltpu.SemaphoreType.DMA for _ in range(_NB)],
        ],
    )
    def emb(idx_hbm, table_hbm, pe_hbm, out_hbm,
            idx_v, pe_v, bufs, gsems, ssems):
        wid = lax.axis_index("s") * _NC + lax.axis_index("c")
        base = wid * rpw
        pltpu.sync_copy(idx_hbm.at[wid], idx_v)
        pltpu.sync_copy(pe_hbm, pe_v)

        def issue_gather(j, b):
            pltpu.async_copy(table_hbm.at[idx_v.at[j]], bufs[b], gsems[b])

        def wait_gather(j, b):
            pltpu.make_async_copy(
                table_hbm.at[idx_v.at[j]], bufs[b], gsems[b]).wait()

        def out_slice(j):
            return out_hbm.at[pl.ds(base + j * _CHUNK, _CHUNK)]

        def issue_store(j, b):
            pltpu.async_copy(bufs[b], out_slice(j), ssems[b])

        def wait_store(j, b):
            pltpu.make_async_copy(bufs[b], out_slice(j), ssems[b]).wait()

        def add_pe(j, b):
            p0 = lax.rem(j * _CHUNK, seq_len)
            buf = bufs[b]

            def row_body(r0, carry):
                rbase = r0 * _RUNROLL
                for u in range(_RUNROLL):
                    for c in range(nvec):
                        sl = pl.ds(c * _LANES, _LANES)
                        plsc.addupdate(buf.at[rbase + u, sl],
                                       pe_v[p0 + rbase + u, sl])
                return carry

            lax.fori_loop(0, _CHUNK // _RUNROLL, row_body, 0)

        for b in range(_NB - 2):
            issue_gather(b, b)

        def body(jj, carry):
            for k in range(_NB):
                j = _NB * jj + k
                bq = (k - 2) % _NB

                @pl.when(j >= 2)
                def _():
                    wait_store(j - 2, bq)

                @pl.when(j + _NB - 2 < nch)
                def _():
                    issue_gather(j + _NB - 2, bq)

                wait_gather(j, k)
                add_pe(j, k)
                issue_store(j, k)
            return carry

        lax.fori_loop(0, nch // _NB, body, 0)

        for b in (_NB - 2, _NB - 1):
            wait_store(nch - _NB + b, b)

    out = emb(idx, token_embeddings, pe2)
    return out.reshape(batch, seq_len, emb_dim)
